# baseline (speedup 1.0000x reference)
import functools

import jax
import jax.numpy as jnp
from jax import lax
from jax.experimental import pallas as pl
from jax.experimental.pallas import tpu as pltpu
from jax.experimental.pallas import tpu_sc as plsc

_N_NODES = 10000
_N_EDGES = 160000
_NC = 2
_NS = 16
_LANES = 16
_CHUNK = 128
_EDGES_PAD = 163840
_N_ACC = 10240


def _mm_body(a_ref, b_ref, o_ref):
    o_ref[...] = lax.dot_general(
        a_ref[...], b_ref[...], (((1,), (0,)), ((), ())),
        preferred_element_type=jnp.float32,
        precision=lax.Precision.HIGHEST).astype(o_ref.dtype)


def _matmul(a, b, bm, out_dtype=jnp.float32):
    m, k = a.shape
    _, n = b.shape
    return pl.pallas_call(
        _mm_body,
        grid=(m // bm,),
        in_specs=[pl.BlockSpec((bm, k), lambda i: (i, 0)),
                  pl.BlockSpec((k, n), lambda i: (0, 0))],
        out_specs=pl.BlockSpec((bm, n), lambda i: (i, 0)),
        out_shape=jax.ShapeDtypeStruct((m, n), out_dtype),
    )(a, b)


def _mid_body(p0_ref, p1_ref, b1_ref, w2_ref, o_ref):
    h = jnp.maximum(p0_ref[...].astype(jnp.float32)
                    + p1_ref[...].astype(jnp.float32) + b1_ref[...], 0.0)
    o_ref[...] = lax.dot_general(
        h, w2_ref[...], (((1,), (0,)), ((), ())),
        preferred_element_type=jnp.float32,
        precision=lax.Precision.HIGHEST)


def _mid_stage(p0, p1, b1, w2p, bm):
    m, k = p0.shape
    n = w2p.shape[1]
    return pl.pallas_call(
        _mid_body,
        grid=(m // bm,),
        in_specs=[pl.BlockSpec((bm, k), lambda i: (i, 0)),
                  pl.BlockSpec((bm, k), lambda i: (i, 0)),
                  pl.BlockSpec((1, k), lambda i: (0, 0)),
                  pl.BlockSpec((k, n), lambda i: (0, 0))],
        out_specs=pl.BlockSpec((bm, n), lambda i: (i, 0)),
        out_shape=jax.ShapeDtypeStruct((m, n), jnp.float32),
    )(p0, p1, b1.reshape(1, k), w2p)


def _final_body(p0_ref, p1_ref, b_ref, o_ref):
    o_ref[...] = p0_ref[...] + p1_ref[...] + b_ref[...]


def _final_stage(p0, p1, btile):
    m, n = p0.shape
    return pl.pallas_call(
        _final_body,
        in_specs=[pl.BlockSpec((m, n), lambda: (0, 0)),
                  pl.BlockSpec((m, n), lambda: (0, 0)),
                  pl.BlockSpec((1, n), lambda: (0, 0))],
        out_specs=pl.BlockSpec((m, n), lambda: (0, 0)),
        out_shape=jax.ShapeDtypeStruct((m, n), jnp.float32),
    )(p0, p1, btile.reshape(1, n))


def _make_spmm(n_nodes, d, n_edges_pad, nbuf, dtype=jnp.float32):
    mesh = plsc.VectorSubcoreMesh(core_axis_name="c", subcore_axis_name="s")
    chunks_total = n_edges_pad // _CHUNK
    cw = chunks_total // (_NC * _NS)
    stripe = n_nodes // _NS
    lanes_v = _LANES * (4 // jnp.dtype(dtype).itemsize)
    nvec = d // lanes_v
    idist = nbuf - 1 if nbuf > 2 else 1
    gdist = nbuf - 2 if nbuf > 2 else 1

    @functools.partial(
        pl.kernel,
        mesh=mesh,
        compiler_params=pltpu.CompilerParams(use_tc_tiling_on_sc=False,
                                             needs_layout_passes=False),
        out_type=jax.ShapeDtypeStruct((_NC, n_nodes, d), dtype),
        scratch_types=[
            pltpu.VMEM((nbuf, 3, _CHUNK), jnp.int32),
            pltpu.VMEM((nbuf, _CHUNK, d), dtype),
            pltpu.VMEM_SHARED((n_nodes, d), dtype),
        ] + [pltpu.SemaphoreType.DMA] * (3 * nbuf),
    )
    def spmm(y_hbm, pk_hbm, out_hbm, ib, rows, acc_sh, *sems):
        c = lax.axis_index("c")
        s = lax.axis_index("s")
        zeros = jnp.zeros((lanes_v,), dtype)
        isem = sems[:nbuf]
        gsem = sems[nbuf:2 * nbuf]
        ssem = sems[2 * nbuf:]

        @pl.loop(0, _CHUNK)
        def _(e):
            for j in range(nvec):
                rows[0, e, pl.ds(j * lanes_v, lanes_v)] = zeros

        base = s * stripe
        nfull = stripe // _CHUNK
        rem = stripe - nfull * _CHUNK

        @pl.loop(0, nfull)
        def _(i):
            pltpu.sync_copy(rows.at[0],
                            acc_sh.at[pl.ds(base + i * _CHUNK, _CHUNK)])
        if rem:
            pltpu.sync_copy(rows.at[0].at[pl.ds(0, rem)],
                            acc_sh.at[pl.ds(base + nfull * _CHUNK, rem)])

        plsc.subcore_barrier()

        cb = (c * _NS + s) * cw

        for j in range(idist):
            pltpu.async_copy(pk_hbm.at[cb + j], ib.at[j], isem[j])
        for j in range(gdist):
            pltpu.make_async_copy(pk_hbm.at[cb + j], ib.at[j], isem[j]).wait()
            pltpu.async_copy(y_hbm.at[ib.at[j].at[0]], rows.at[j], gsem[j])

        @pl.loop(0, cw // nbuf)
        def _(t):
            for b in range(nbuf):
                i = t * nbuf + b
                ib_b, rows_b = ib.at[b], rows.at[b]
                bprev = (b - 1) % nbuf
                bg = (b + gdist) % nbuf

                @pl.when(i >= 1)
                def _():
                    pltpu.make_async_copy(
                        rows.at[bprev], acc_sh.at[ib.at[bprev].at[1]],
                        ssem[bprev]).wait()

                @pl.when(i + idist < cw)
                def _():
                    pltpu.async_copy(pk_hbm.at[cb + i + idist], ib.at[bprev],
                                     isem[bprev])

                @pl.when(i + gdist < cw)
                def _():
                    pltpu.make_async_copy(pk_hbm.at[cb + i + gdist],
                                          ib.at[bg], isem[bg]).wait()
                    pltpu.async_copy(y_hbm.at[ib.at[bg].at[0]], rows.at[bg],
                                     gsem[bg])

                pltpu.make_async_copy(
                    y_hbm.at[ib_b.at[0]], rows_b, gsem[b]).wait()

                @pl.loop(0, _CHUNK // _LANES)
                def _(g):
                    wvec = plsc.bitcast(ib_b[2, pl.ds(g * _LANES, _LANES)],
                                        jnp.float32)
                    for k in range(_LANES):
                        wf = jnp.full((_LANES,), wvec[k], jnp.float32)
                        if dtype == jnp.bfloat16:
                            wb = plsc.pack(wf, wf,
                                           format=plsc.PackFormat.INTERLEAVED)
                        else:
                            wb = wf
                        e = g * _LANES + k
                        for j in range(nvec):
                            sl = pl.ds(j * lanes_v, lanes_v)
                            rows_b[e, sl] = rows_b[e, sl] * wb

                pltpu.async_copy(rows_b, acc_sh.at[ib_b.at[1]], ssem[b],
                                 add=True)

        lb = (cw - 1) % nbuf
        pltpu.make_async_copy(rows.at[lb], acc_sh.at[ib.at[lb].at[1]],
                              ssem[lb]).wait()

        plsc.subcore_barrier()

        pltpu.sync_copy(acc_sh.at[pl.ds(base, stripe)],
                        out_hbm.at[c].at[pl.ds(base, stripe)])

    return spmm


_spmm_128 = _make_spmm(_N_ACC, 128, _EDGES_PAD, 4, jnp.bfloat16)
_spmm_16 = _make_spmm(_N_ACC, 16, _EDGES_PAD, 4)


def kernel(x, edge_index, edge_weight, W1, b1, W2, b2):
    src = edge_index[0].astype(jnp.int32)
    dst = edge_index[1].astype(jnp.int32)
    w = edge_weight.astype(jnp.float32)

    pad = _EDGES_PAD - _N_EDGES
    spread = jnp.arange(pad, dtype=jnp.int32) % _N_NODES
    src = jnp.concatenate([src, spread])
    dst = jnp.concatenate([dst, spread])
    w = jnp.concatenate([w, jnp.zeros((pad,), jnp.float32)])
    nchunks = _EDGES_PAD // _CHUNK
    packed = jnp.stack([src.reshape(nchunks, _CHUNK),
                        dst.reshape(nchunks, _CHUNK),
                        lax.bitcast_convert_type(w, jnp.int32)
                           .reshape(nchunks, _CHUNK)], axis=1)

    y1 = _matmul(x, W1, 1000, jnp.bfloat16)

    p = _spmm_128(y1, packed)

    w2p = jnp.pad(W2, ((0, 0), (0, 16 - W2.shape[1])))
    y2 = _mid_stage(p[0], p[1], b1, w2p, 1024)

    q = _spmm_16(y2, packed)

    b2t = jnp.tile(jnp.pad(b2, (0, 16 - b2.shape[0])), 8)
    out16 = _final_stage(q[0].reshape(1280, 128), q[1].reshape(1280, 128), b2t)
    return out16.reshape(_N_ACC, 16)[:_N_NODES, :2]


# speedup vs baseline: 1.1789x; 1.1789x over previous
import functools

import jax
import jax.numpy as jnp
from jax import lax
from jax.experimental import pallas as pl
from jax.experimental.pallas import tpu as pltpu
from jax.experimental.pallas import tpu_sc as plsc

_N_NODES = 10000
_N_EDGES = 160000
_NC = 2
_NS = 16
_LANES = 16
_CHUNK = 128
_EDGES_PAD = 163840
_N_ACC = 10240


def _mm_body(a_ref, b_ref, o_ref):
    o_ref[...] = lax.dot_general(
        a_ref[...], b_ref[...], (((1,), (0,)), ((), ())),
        preferred_element_type=jnp.float32).astype(o_ref.dtype)


def _matmul(a, b, bm, out_dtype=jnp.float32):
    m, k = a.shape
    _, n = b.shape
    return pl.pallas_call(
        _mm_body,
        grid=(m // bm,),
        in_specs=[pl.BlockSpec((bm, k), lambda i: (i, 0)),
                  pl.BlockSpec((k, n), lambda i: (0, 0))],
        out_specs=pl.BlockSpec((bm, n), lambda i: (i, 0)),
        out_shape=jax.ShapeDtypeStruct((m, n), out_dtype),
    )(a, b)


def _mid_body(p_ref, b1_ref, w2_ref, o_ref):
    h = jnp.maximum(p_ref[0].astype(jnp.float32)
                    + p_ref[1].astype(jnp.float32) + b1_ref[...], 0.0)
    o_ref[...] = lax.dot_general(
        h, w2_ref[...], (((1,), (0,)), ((), ())),
        preferred_element_type=jnp.float32)


def _mid_stage(p, b1, w2p, bm):
    _, m, k = p.shape
    n = w2p.shape[1]
    return pl.pallas_call(
        _mid_body,
        grid=(m // bm,),
        in_specs=[pl.BlockSpec((2, bm, k), lambda i: (0, i, 0)),
                  pl.BlockSpec((1, k), lambda i: (0, 0)),
                  pl.BlockSpec((k, n), lambda i: (0, 0))],
        out_specs=pl.BlockSpec((bm, n), lambda i: (i, 0)),
        out_shape=jax.ShapeDtypeStruct((m, n), jnp.float32),
    )(p, b1.reshape(1, k), w2p)


def _final_body(q_ref, b_ref, o_ref):
    o_ref[...] = q_ref[0] + q_ref[1] + b_ref[...]


def _final_stage(q, btile):
    _, m, n = q.shape
    return pl.pallas_call(
        _final_body,
        in_specs=[pl.BlockSpec((2, m, n), lambda: (0, 0, 0)),
                  pl.BlockSpec((1, n), lambda: (0, 0))],
        out_specs=pl.BlockSpec((m, n), lambda: (0, 0)),
        out_shape=jax.ShapeDtypeStruct((m, n), jnp.float32),
    )(q, btile.reshape(1, n))


def _make_spmm(n_nodes, d, n_edges_pad, nbuf, dtype=jnp.float32):
    mesh = plsc.VectorSubcoreMesh(core_axis_name="c", subcore_axis_name="s")
    chunks_total = n_edges_pad // _CHUNK
    cw = chunks_total // (_NC * _NS)
    stripe = n_nodes // _NS
    lanes_v = _LANES * (4 // jnp.dtype(dtype).itemsize)
    nvec = d // lanes_v
    idist = nbuf - 1 if nbuf > 2 else 1
    gdist = nbuf - 2 if nbuf > 2 else 1

    @functools.partial(
        pl.kernel,
        mesh=mesh,
        compiler_params=pltpu.CompilerParams(use_tc_tiling_on_sc=False,
                                             needs_layout_passes=False),
        out_type=jax.ShapeDtypeStruct((_NC, n_nodes, d), dtype),
        scratch_types=[
            pltpu.VMEM((nbuf, 3, _CHUNK), jnp.int32),
            pltpu.VMEM((nbuf, _CHUNK, d), dtype),
            pltpu.VMEM_SHARED((n_nodes, d), dtype),
        ] + [pltpu.SemaphoreType.DMA] * (3 * nbuf),
    )
    def spmm(y_hbm, pk_hbm, out_hbm, ib, rows, acc_sh, *sems):
        c = lax.axis_index("c")
        s = lax.axis_index("s")
        zeros = jnp.zeros((lanes_v,), dtype)
        isem = sems[:nbuf]
        gsem = sems[nbuf:2 * nbuf]
        ssem = sems[2 * nbuf:]

        @pl.loop(0, _CHUNK)
        def _(e):
            for j in range(nvec):
                rows[0, e, pl.ds(j * lanes_v, lanes_v)] = zeros

        base = s * stripe
        nfull = stripe // _CHUNK
        rem = stripe - nfull * _CHUNK

        @pl.loop(0, nfull)
        def _(i):
            pltpu.sync_copy(rows.at[0],
                            acc_sh.at[pl.ds(base + i * _CHUNK, _CHUNK)])
        if rem:
            pltpu.sync_copy(rows.at[0].at[pl.ds(0, rem)],
                            acc_sh.at[pl.ds(base + nfull * _CHUNK, rem)])

        plsc.subcore_barrier()

        cb = (c * _NS + s) * cw

        for j in range(idist):
            pltpu.async_copy(pk_hbm.at[cb + j], ib.at[j], isem[j])
        for j in range(gdist):
            pltpu.make_async_copy(pk_hbm.at[cb + j], ib.at[j], isem[j]).wait()
            pltpu.async_copy(y_hbm.at[ib.at[j].at[0]], rows.at[j], gsem[j])

        @pl.loop(0, cw // nbuf)
        def _(t):
            for b in range(nbuf):
                i = t * nbuf + b
                ib_b, rows_b = ib.at[b], rows.at[b]
                bprev = (b - 1) % nbuf
                bg = (b + gdist) % nbuf

                @pl.when(i >= 1)
                def _():
                    pltpu.make_async_copy(
                        rows.at[bprev], acc_sh.at[ib.at[bprev].at[1]],
                        ssem[bprev]).wait()

                @pl.when(i + idist < cw)
                def _():
                    pltpu.async_copy(pk_hbm.at[cb + i + idist], ib.at[bprev],
                                     isem[bprev])

                @pl.when(i + gdist < cw)
                def _():
                    pltpu.make_async_copy(pk_hbm.at[cb + i + gdist],
                                          ib.at[bg], isem[bg]).wait()
                    pltpu.async_copy(y_hbm.at[ib.at[bg].at[0]], rows.at[bg],
                                     gsem[bg])

                pltpu.make_async_copy(
                    y_hbm.at[ib_b.at[0]], rows_b, gsem[b]).wait()

                @pl.loop(0, _CHUNK // _LANES)
                def _(g):
                    wvec = plsc.bitcast(ib_b[2, pl.ds(g * _LANES, _LANES)],
                                        jnp.float32)
                    for k in range(_LANES):
                        wf = jnp.full((_LANES,), wvec[k], jnp.float32)
                        if dtype == jnp.bfloat16:
                            wb = plsc.pack(wf, wf,
                                           format=plsc.PackFormat.INTERLEAVED)
                        else:
                            wb = wf
                        e = g * _LANES + k
                        for j in range(nvec):
                            sl = pl.ds(j * lanes_v, lanes_v)
                            rows_b[e, sl] = rows_b[e, sl] * wb

                pltpu.async_copy(rows_b, acc_sh.at[ib_b.at[1]], ssem[b],
                                 add=True)

        lb = (cw - 1) % nbuf
        pltpu.make_async_copy(rows.at[lb], acc_sh.at[ib.at[lb].at[1]],
                              ssem[lb]).wait()

        plsc.subcore_barrier()

        pltpu.sync_copy(acc_sh.at[pl.ds(base, stripe)],
                        out_hbm.at[c].at[pl.ds(base, stripe)])

    return spmm


_spmm_128 = _make_spmm(_N_ACC, 128, _EDGES_PAD, 5, jnp.bfloat16)
_spmm_16 = _make_spmm(_N_ACC, 16, _EDGES_PAD, 5)


def kernel(x, edge_index, edge_weight, W1, b1, W2, b2):
    src = edge_index[0].astype(jnp.int32)
    dst = edge_index[1].astype(jnp.int32)
    w = edge_weight.astype(jnp.float32)

    pad = _EDGES_PAD - _N_EDGES
    spread = jnp.arange(pad, dtype=jnp.int32) % _N_NODES
    src = jnp.concatenate([src, spread])
    dst = jnp.concatenate([dst, spread])
    w = jnp.concatenate([w, jnp.zeros((pad,), jnp.float32)])
    nchunks = _EDGES_PAD // _CHUNK
    packed = jnp.stack([src.reshape(nchunks, _CHUNK),
                        dst.reshape(nchunks, _CHUNK),
                        lax.bitcast_convert_type(w, jnp.int32)
                           .reshape(nchunks, _CHUNK)], axis=1)

    y1 = _matmul(x, W1, 1000, jnp.bfloat16)

    p = _spmm_128(y1, packed)

    w2p = jnp.pad(W2, ((0, 0), (0, 16 - W2.shape[1])))
    y2 = _mid_stage(p, b1, w2p, 1024)

    q = _spmm_16(y2, packed)

    b2t = jnp.tile(jnp.pad(b2, (0, 16 - b2.shape[0])), 8)
    out16 = _final_stage(q.reshape(2, 1280, 128), b2t)
    return out16.reshape(_N_ACC, 16)[:_N_NODES, :2]


# speedup vs baseline: 1.2151x; 1.0307x over previous
import functools

import jax
import jax.numpy as jnp
from jax import lax
from jax.experimental import pallas as pl
from jax.experimental.pallas import tpu as pltpu
from jax.experimental.pallas import tpu_sc as plsc

_N_NODES = 10000
_N_EDGES = 160000
_NC = 2
_NS = 16
_LANES = 16
_CHUNK = 128
_EDGES_PAD = 163840
_N_ACC = 10240


def _mm_body(a_ref, b_ref, o_ref):
    o_ref[...] = lax.dot_general(
        a_ref[...], b_ref[...], (((1,), (0,)), ((), ())),
        preferred_element_type=jnp.float32).astype(o_ref.dtype)


def _matmul(a, b, bm, out_dtype=jnp.float32):
    m, k = a.shape
    _, n = b.shape
    return pl.pallas_call(
        _mm_body,
        grid=(m // bm,),
        in_specs=[pl.BlockSpec((bm, k), lambda i: (i, 0)),
                  pl.BlockSpec((k, n), lambda i: (0, 0))],
        out_specs=pl.BlockSpec((bm, n), lambda i: (i, 0)),
        out_shape=jax.ShapeDtypeStruct((m, n), out_dtype),
    )(a, b)


def _mid_body(p_ref, b1_ref, w2_ref, o_ref):
    h = jnp.maximum(p_ref[0].astype(jnp.float32)
                    + p_ref[1].astype(jnp.float32) + b1_ref[...], 0.0)
    o_ref[...] = lax.dot_general(
        h, w2_ref[...], (((1,), (0,)), ((), ())),
        preferred_element_type=jnp.float32)


def _mid_stage(p, b1, w2p, bm):
    _, m, k = p.shape
    n = w2p.shape[1]
    return pl.pallas_call(
        _mid_body,
        grid=(m // bm,),
        in_specs=[pl.BlockSpec((2, bm, k), lambda i: (0, i, 0)),
                  pl.BlockSpec((1, k), lambda i: (0, 0)),
                  pl.BlockSpec((k, n), lambda i: (0, 0))],
        out_specs=pl.BlockSpec((bm, n), lambda i: (i, 0)),
        out_shape=jax.ShapeDtypeStruct((m, n), jnp.float32),
    )(p, b1.reshape(1, k), w2p)


def _final_body(q_ref, b_ref, o_ref):
    o_ref[...] = q_ref[0] + q_ref[1] + b_ref[...]


def _final_stage(q, btile):
    _, m, n = q.shape
    return pl.pallas_call(
        _final_body,
        in_specs=[pl.BlockSpec((2, m, n), lambda: (0, 0, 0)),
                  pl.BlockSpec((1, n), lambda: (0, 0))],
        out_specs=pl.BlockSpec((m, n), lambda: (0, 0)),
        out_shape=jax.ShapeDtypeStruct((m, n), jnp.float32),
    )(q, btile.reshape(1, n))


def _make_spmm(n_nodes, d, n_edges_pad, nbuf, dtype=jnp.float32):
    mesh = plsc.VectorSubcoreMesh(core_axis_name="c", subcore_axis_name="s")
    chunks_total = n_edges_pad // _CHUNK
    cw = chunks_total // (_NC * _NS)
    stripe = n_nodes // _NS
    lanes_v = _LANES * (4 // jnp.dtype(dtype).itemsize)
    nvec = d // lanes_v
    idist = nbuf - 1 if nbuf > 2 else 1
    gdist = nbuf - 2 if nbuf > 2 else 1

    @functools.partial(
        pl.kernel,
        mesh=mesh,
        compiler_params=pltpu.CompilerParams(use_tc_tiling_on_sc=False,
                                             needs_layout_passes=False),
        out_type=jax.ShapeDtypeStruct((_NC, n_nodes, d), dtype),
        scratch_types=[
            pltpu.VMEM((nbuf, 3, _CHUNK), jnp.int32),
            pltpu.VMEM((nbuf, _CHUNK, d), dtype),
            pltpu.VMEM_SHARED((n_nodes, d), dtype),
        ] + [pltpu.SemaphoreType.DMA] * (3 * nbuf),
    )
    def spmm(y_hbm, pk_hbm, out_hbm, ib, rows, acc_sh, *sems):
        c = lax.axis_index("c")
        s = lax.axis_index("s")
        zeros = jnp.zeros((lanes_v,), dtype)
        isem = sems[:nbuf]
        gsem = sems[nbuf:2 * nbuf]
        ssem = sems[2 * nbuf:]

        @pl.loop(0, _CHUNK)
        def _(e):
            for j in range(nvec):
                rows[0, e, pl.ds(j * lanes_v, lanes_v)] = zeros

        base = s * stripe
        nfull = stripe // _CHUNK
        rem = stripe - nfull * _CHUNK

        @pl.loop(0, nfull)
        def _(i):
            pltpu.sync_copy(rows.at[0],
                            acc_sh.at[pl.ds(base + i * _CHUNK, _CHUNK)])
        if rem:
            pltpu.sync_copy(rows.at[0].at[pl.ds(0, rem)],
                            acc_sh.at[pl.ds(base + nfull * _CHUNK, rem)])

        plsc.subcore_barrier()

        cb = (c * _NS + s) * cw

        def idx_fetch(chunk, slot):
            for r in range(3):
                pltpu.async_copy(pk_hbm.at[r].at[chunk], ib.at[slot].at[r],
                                 isem[slot])

        def idx_wait(chunk, slot):
            for r in range(3):
                pltpu.make_async_copy(pk_hbm.at[r].at[chunk],
                                      ib.at[slot].at[r], isem[slot]).wait()

        for j in range(idist):
            idx_fetch(cb + j, j)
        for j in range(gdist):
            idx_wait(cb + j, j)
            pltpu.async_copy(y_hbm.at[ib.at[j].at[0]], rows.at[j], gsem[j])

        @pl.loop(0, cw // nbuf)
        def _(t):
            for b in range(nbuf):
                i = t * nbuf + b
                ib_b, rows_b = ib.at[b], rows.at[b]
                bprev = (b - 1) % nbuf
                bg = (b + gdist) % nbuf

                @pl.when(i >= 1)
                def _():
                    pltpu.make_async_copy(
                        rows.at[bprev], acc_sh.at[ib.at[bprev].at[1]],
                        ssem[bprev]).wait()

                @pl.when(i + idist < cw)
                def _():
                    idx_fetch(cb + i + idist, bprev)

                @pl.when(i + gdist < cw)
                def _():
                    idx_wait(cb + i + gdist, bg)
                    pltpu.async_copy(y_hbm.at[ib.at[bg].at[0]], rows.at[bg],
                                     gsem[bg])

                pltpu.make_async_copy(
                    y_hbm.at[ib_b.at[0]], rows_b, gsem[b]).wait()

                @pl.loop(0, _CHUNK // _LANES)
                def _(g):
                    wvec = plsc.bitcast(ib_b[2, pl.ds(g * _LANES, _LANES)],
                                        jnp.float32)
                    for k in range(_LANES):
                        wf = jnp.full((_LANES,), wvec[k], jnp.float32)
                        if dtype == jnp.bfloat16:
                            wb = plsc.pack(wf, wf,
                                           format=plsc.PackFormat.INTERLEAVED)
                        else:
                            wb = wf
                        e = g * _LANES + k
                        for j in range(nvec):
                            sl = pl.ds(j * lanes_v, lanes_v)
                            rows_b[e, sl] = rows_b[e, sl] * wb

                pltpu.async_copy(rows_b, acc_sh.at[ib_b.at[1]], ssem[b],
                                 add=True)

        lb = (cw - 1) % nbuf
        pltpu.make_async_copy(rows.at[lb], acc_sh.at[ib.at[lb].at[1]],
                              ssem[lb]).wait()

        plsc.subcore_barrier()

        pltpu.sync_copy(acc_sh.at[pl.ds(base, stripe)],
                        out_hbm.at[c].at[pl.ds(base, stripe)])

    return spmm


_spmm_128 = _make_spmm(_N_ACC, 128, _EDGES_PAD, 5, jnp.bfloat16)
_spmm_16 = _make_spmm(_N_ACC, 16, _EDGES_PAD, 8)


def kernel(x, edge_index, edge_weight, W1, b1, W2, b2):
    src = edge_index[0].astype(jnp.int32)
    dst = edge_index[1].astype(jnp.int32)
    w = edge_weight.astype(jnp.float32)

    pad = _EDGES_PAD - _N_EDGES
    spread = jnp.arange(pad, dtype=jnp.int32) % _N_NODES
    src = jnp.concatenate([src, spread])
    dst = jnp.concatenate([dst, spread])
    w = jnp.concatenate([w, jnp.zeros((pad,), jnp.float32)])
    nchunks = _EDGES_PAD // _CHUNK
    packed = jnp.stack([src.reshape(nchunks, _CHUNK),
                        dst.reshape(nchunks, _CHUNK),
                        lax.bitcast_convert_type(w, jnp.int32)
                           .reshape(nchunks, _CHUNK)], axis=0)

    y1 = _matmul(x, W1, 2000, jnp.bfloat16)

    p = _spmm_128(y1, packed)

    w2p = jnp.pad(W2, ((0, 0), (0, 16 - W2.shape[1])))
    y2 = _mid_stage(p, b1, w2p, 2048)

    q = _spmm_16(y2, packed)

    b2t = jnp.tile(jnp.pad(b2, (0, 16 - b2.shape[0])), 8)
    out16 = _final_stage(q.reshape(2, 1280, 128), b2t)
    return out16.reshape(_N_ACC, 16)[:_N_NODES, :2]
